# baseline (speedup 1.0000x reference)
import jax
import jax.numpy as jnp
from jax import lax
from jax.experimental import pallas as pl
from jax.experimental.pallas import tpu as pltpu
from jax.experimental.pallas import tpu_sc as plsc

N_NODES = 10000
N_EDGES = 320000
D = 128
D_EDGE = 16

NC = 2
NS = 16
NW = NC * NS

E_PER_W = N_EDGES // NW
CHUNK = 40
N_CHUNKS = E_PER_W // CHUNK
ROWS_LO = 624
ROWS_HI = 640


def _bf16_bits(v):
    u = jax.lax.bitcast_convert_type(v, jnp.uint32)
    return (u + 0x7FFF + ((u >> 16) & 1)) >> 16


def _eproj_body(ea_ref, w_ref, b_ref, out_ref):
    ea = ea_ref[0]
    full = jnp.dot(ea, w_ref[...], preferred_element_type=jnp.float32) + b_ref[...]
    u = _bf16_bits(full[:, :D // 2]) | (_bf16_bits(full[:, D // 2:]) << 16)
    u3 = u.reshape(N_CHUNKS // 2, 2, CHUNK, D // 2)
    word = jnp.concatenate([u3[:, 0], u3[:, 1]], axis=-1)
    out_ref[0] = jax.lax.bitcast_convert_type(word, jnp.int32)


def _eproj(edge_attr, W_edge, b_edge):
    return pl.pallas_call(
        _eproj_body,
        grid=(NW,),
        in_specs=[
            pl.BlockSpec((1, E_PER_W, D_EDGE), lambda i: (i, 0, 0)),
            pl.BlockSpec((D_EDGE, D), lambda i: (0, 0)),
            pl.BlockSpec((1, D), lambda i: (0, 0)),
        ],
        out_specs=pl.BlockSpec((1, N_CHUNKS // 2, CHUNK, D), lambda i: (i, 0, 0, 0)),
        out_shape=jax.ShapeDtypeStruct((NW, N_CHUNKS // 2, CHUNK, D), jnp.int32),
    )(edge_attr.reshape(NW, E_PER_W, D_EDGE), W_edge, b_edge.reshape(1, D))


def _sc_body(x_hbm, ep_hbm, pidx_hbm, out_hbm,
             pidx_v, si0, si1, si2, si3, di0, di1, di2, di3,
             xr0, xr1, eb0, eb1, msg0, msg1,
             aggr_sp, g0, g1, e0, e1, s0, s1):
    cid = lax.axis_index("c")
    sid = lax.axis_index("s")
    wid = cid * NS + sid

    xr = (xr0, xr1)
    eb = (eb0, eb1)
    msg = (msg0, msg1)
    sidx = (si0, si1, si2, si3)
    didx = (di0, di1, di2, di3)
    gsem = (g0, g1)
    esem = (e0, e1)
    ssem = (s0, s1)

    pltpu.sync_copy(pidx_hbm.at[wid], pidx_v)

    def unpack_idx(t, q):
        for off in (0, 16, 24):
            p = pidx_v[pl.ds(t * CHUNK + off, 16)]
            sidx[q][pl.ds(off, 16)] = p & 0x3FFF
            didx[q][pl.ds(off, 16)] = p >> 14

    def issue_gather(j, b, q):
        pltpu.async_copy(x_hbm.at[sidx[q]], xr[b], gsem[b])

    def wait_gather(b):
        pltpu.make_async_copy(x_hbm.at[sidx[0]], xr[b], gsem[b]).wait()

    def issue_ep_pair(p, pb):
        pltpu.async_copy(ep_hbm.at[wid, p], eb[pb], esem[pb])

    def wait_ep_pair(pb):
        pltpu.make_async_copy(ep_hbm.at[wid, 0], eb[pb], esem[pb]).wait()

    def issue_scatter(b, q):
        pltpu.async_copy(msg[b], aggr_sp.at[didx[q]], ssem[b], add=True)

    def wait_scatter(b):
        pltpu.make_async_copy(msg[b], aggr_sp.at[didx[0]], ssem[b]).wait()

    def compute(b, pb, half):
        xrb, ebb, msgb = xr[b], eb[pb], msg[b]
        base = half * (D // 2)

        def rows(r8, _):
            for u in range(8):
                r = r8 * 8 + u
                for g in range(4):
                    packed = ebb[r, pl.ds(base + g * 16, 16)]
                    lo, hi = plsc.unpack(
                        plsc.bitcast(packed, jnp.bfloat16),
                        format=plsc.PackFormat.INTERLEAVED)
                    sa = pl.ds(g * 16, 16)
                    sb = pl.ds(64 + g * 16, 16)
                    msgb[r, sa] = jnp.maximum(xrb[r, sa] + lo, 0.0)
                    msgb[r, sb] = jnp.maximum(xrb[r, sb] + hi, 0.0)
            return 0

        lax.fori_loop(0, CHUNK // 8, rows, 0)

    for t in range(4):
        unpack_idx(t, t)
    issue_gather(0, 0, 0)
    issue_ep_pair(0, 0)
    issue_gather(1, 1, 1)
    issue_ep_pair(1, 1)

    zeros16 = jnp.zeros((16,), jnp.float32)

    def zrow(r, _):
        for c in range(D // 16):
            msg0[r, pl.ds(c * 16, 16)] = zeros16
        return 0

    lax.fori_loop(0, CHUNK, zrow, 0)
    zbase = sid * ROWS_LO

    def zcopy(k, _):
        pltpu.sync_copy(msg0, aggr_sp.at[pl.ds(zbase + CHUNK * k, CHUNK)])
        return 0

    lax.fori_loop(0, 15, zcopy, 0)

    @pl.when(sid < NS - 1)
    def _():
        pltpu.sync_copy(msg0.at[pl.ds(0, 24)],
                        aggr_sp.at[pl.ds(zbase + 600, 24)])

    @pl.when(sid == NS - 1)
    def _():
        pltpu.sync_copy(msg0, aggr_sp.at[pl.ds(9960, CHUNK)])

    plsc.subcore_barrier()

    def step(j, b, q, pb, half, do_wait_scatter, do_issue, ep_wait, ep_issue):
        wait_gather(b)
        if ep_wait:
            wait_ep_pair(pb)
        if do_wait_scatter:
            wait_scatter(b)
        compute(b, pb, half)
        issue_scatter(b, q)
        if do_issue:
            q2 = (q + 2) % 4
            unpack_idx(j + 2, q2)
            issue_gather(j + 2, b, q2)
        if ep_issue:
            issue_ep_pair(j // 2 + 1, 1 - pb)

    wait_gather(0)
    wait_ep_pair(0)
    compute(0, 0, 0)
    issue_scatter(0, 0)
    issue_gather(2, 0, 2)

    wait_gather(1)
    compute(1, 0, 1)
    issue_scatter(1, 1)
    issue_gather(3, 1, 3)

    step(2, 0, 2, 1, 0, True, True, True, True)
    step(3, 1, 3, 1, 1, True, True, False, False)

    def quad(i, _):
        j = 4 * i
        step(j + 0, 0, 0, 0, 0, True, True, True, True)
        step(j + 1, 1, 1, 0, 1, True, True, False, False)
        step(j + 2, 0, 2, 1, 0, True, True, True, True)
        step(j + 3, 1, 3, 1, 1, True, True, False, False)
        return 0

    lax.fori_loop(1, 61, quad, 0)

    step(244, 0, 0, 0, 0, True, True, True, True)
    step(245, 1, 1, 0, 1, True, True, False, False)
    step(246, 0, 2, 1, 0, True, True, True, True)
    step(247, 1, 3, 1, 1, True, True, False, False)
    step(248, 0, 0, 0, 0, True, False, True, False)
    step(249, 1, 1, 0, 1, True, False, False, False)
    wait_scatter(0)
    wait_scatter(1)

    plsc.subcore_barrier()

    @pl.when(sid < NS - 1)
    def _():
        pltpu.sync_copy(aggr_sp.at[pl.ds(zbase, ROWS_LO)],
                        out_hbm.at[cid, pl.ds(zbase, ROWS_LO)])

    @pl.when(sid == NS - 1)
    def _():
        pltpu.sync_copy(aggr_sp.at[pl.ds((NS - 1) * ROWS_LO, ROWS_HI)],
                        out_hbm.at[cid, pl.ds((NS - 1) * ROWS_LO, ROWS_HI)])


def _aggregate_sc(x, e_proj, pidx):
    mesh = plsc.VectorSubcoreMesh(core_axis_name="c", subcore_axis_name="s")
    f = pl.kernel(
        _sc_body,
        out_type=jax.ShapeDtypeStruct((NC, N_NODES, D), jnp.float32),
        mesh=mesh,
        compiler_params=pltpu.CompilerParams(needs_layout_passes=False),
        scratch_types=[
            pltpu.VMEM((E_PER_W,), jnp.int32),
            pltpu.VMEM((CHUNK,), jnp.int32),
            pltpu.VMEM((CHUNK,), jnp.int32),
            pltpu.VMEM((CHUNK,), jnp.int32),
            pltpu.VMEM((CHUNK,), jnp.int32),
            pltpu.VMEM((CHUNK,), jnp.int32),
            pltpu.VMEM((CHUNK,), jnp.int32),
            pltpu.VMEM((CHUNK,), jnp.int32),
            pltpu.VMEM((CHUNK,), jnp.int32),
            pltpu.VMEM((CHUNK, D), jnp.float32),
            pltpu.VMEM((CHUNK, D), jnp.float32),
            pltpu.VMEM((CHUNK, D), jnp.int32),
            pltpu.VMEM((CHUNK, D), jnp.int32),
            pltpu.VMEM((CHUNK, D), jnp.float32),
            pltpu.VMEM((CHUNK, D), jnp.float32),
            pltpu.VMEM_SHARED((N_NODES, D), jnp.float32),
            pltpu.SemaphoreType.DMA,
            pltpu.SemaphoreType.DMA,
            pltpu.SemaphoreType.DMA,
            pltpu.SemaphoreType.DMA,
            pltpu.SemaphoreType.DMA,
            pltpu.SemaphoreType.DMA,
        ],
    )
    pidx2 = pidx.reshape(NW, E_PER_W)
    return f(x, e_proj, pidx2)


def _mlp_body(x_ref, a0_ref, a1_ref, w1_ref, b1_ref, w2_ref, b2_ref, out_ref):
    h = x_ref[...] + a0_ref[...] + a1_ref[...]
    h = jnp.dot(h, w1_ref[...], preferred_element_type=jnp.float32) + b1_ref[...]
    h = jnp.maximum(h, 0.0)
    out_ref[...] = (
        jnp.dot(h, w2_ref[...], preferred_element_type=jnp.float32) + b2_ref[...]
    )


def _mlp(x, a0, a1, W1, b1, W2, b2):
    BN = 2000
    full = lambda i: (0, 0)
    return pl.pallas_call(
        _mlp_body,
        grid=(N_NODES // BN,),
        in_specs=[
            pl.BlockSpec((BN, D), lambda i: (i, 0)),
            pl.BlockSpec((BN, D), lambda i: (i, 0)),
            pl.BlockSpec((BN, D), lambda i: (i, 0)),
            pl.BlockSpec((D, D), full),
            pl.BlockSpec((1, D), full),
            pl.BlockSpec((D, D), full),
            pl.BlockSpec((1, D), full),
        ],
        out_specs=pl.BlockSpec((BN, D), lambda i: (i, 0)),
        out_shape=jax.ShapeDtypeStruct((N_NODES, D), jnp.float32),
    )(x, a0, a1, W1, b1.reshape(1, D), W2, b2.reshape(1, D))


def kernel(x, edge_index, edge_attr, W_edge, b_edge, W1, b1, W2, b2):
    x = x.astype(jnp.float32)
    edge_attr = edge_attr.astype(jnp.float32)
    ei = edge_index.astype(jnp.int32)
    src, dst = ei[0], ei[1]
    pidx = src | (dst << 14)

    e_proj = _eproj(edge_attr, W_edge, b_edge)
    aggr = _aggregate_sc(x, e_proj, pidx)
    return _mlp(x, aggr[0], aggr[1], W1, b1, W2, b2)


# speedup vs baseline: 1.0149x; 1.0149x over previous
import jax
import jax.numpy as jnp
from jax import lax
from jax.experimental import pallas as pl
from jax.experimental.pallas import tpu as pltpu
from jax.experimental.pallas import tpu_sc as plsc

N_NODES = 10000
N_EDGES = 320000
D = 128
D_EDGE = 16

NC = 2
NS = 16
NW = NC * NS

E_PER_W = N_EDGES // NW
CHUNK = 40
N_CHUNKS = E_PER_W // CHUNK
ROWS_LO = 624
ROWS_HI = 640


def _bf16_bits(v):
    u = jax.lax.bitcast_convert_type(v, jnp.uint32)
    return (u + 0x7FFF + ((u >> 16) & 1)) >> 16


def _eproj_body(ea_ref, w_ref, b_ref, out_ref):
    ea = ea_ref[0]
    full = jnp.dot(ea, w_ref[...], preferred_element_type=jnp.float32) + b_ref[...]
    u = _bf16_bits(full[:, :D // 2]) | (_bf16_bits(full[:, D // 2:]) << 16)
    u3 = u.reshape(N_CHUNKS // 2, 2, CHUNK, D // 2)
    word = jnp.concatenate([u3[:, 0], u3[:, 1]], axis=-1)
    out_ref[0] = jax.lax.bitcast_convert_type(word, jnp.int32)


def _eproj(edge_attr, W_edge, b_edge):
    return pl.pallas_call(
        _eproj_body,
        grid=(NW,),
        in_specs=[
            pl.BlockSpec((1, E_PER_W, D_EDGE), lambda i: (i, 0, 0)),
            pl.BlockSpec((D_EDGE, D), lambda i: (0, 0)),
            pl.BlockSpec((1, D), lambda i: (0, 0)),
        ],
        out_specs=pl.BlockSpec((1, N_CHUNKS // 2, CHUNK, D), lambda i: (i, 0, 0, 0)),
        out_shape=jax.ShapeDtypeStruct((NW, N_CHUNKS // 2, CHUNK, D), jnp.int32),
    )(edge_attr.reshape(NW, E_PER_W, D_EDGE), W_edge, b_edge.reshape(1, D))


def _sc_body(x_hbm, ep_hbm, pidx_hbm, out_hbm,
             pidx_v, si0, si1, si2, si3, di0, di1, di2, di3,
             xr0, xr1, eb0, eb1, msg0, msg1,
             aggr_sp, g0, g1, e0, e1, s0, s1):
    cid = lax.axis_index("c")
    sid = lax.axis_index("s")
    wid = cid * NS + sid

    xr = (xr0, xr1)
    eb = (eb0, eb1)
    msg = (msg0, msg1)
    sidx = (si0, si1, si2, si3)
    didx = (di0, di1, di2, di3)
    gsem = (g0, g1)
    esem = (e0, e1)
    ssem = (s0, s1)

    pltpu.sync_copy(pidx_hbm.at[wid], pidx_v)

    def unpack_idx(t, q):
        for off in (0, 16, 24):
            p = pidx_v[pl.ds(t * CHUNK + off, 16)]
            sidx[q][pl.ds(off, 16)] = p & 0x3FFF
            didx[q][pl.ds(off, 16)] = p >> 14

    def issue_gather(j, b, q):
        pltpu.async_copy(x_hbm.at[sidx[q]], xr[b], gsem[b])

    def wait_gather(b):
        pltpu.make_async_copy(x_hbm.at[sidx[0]], xr[b], gsem[b]).wait()

    def issue_ep_pair(p, pb):
        pltpu.async_copy(ep_hbm.at[wid, p], eb[pb], esem[pb])

    def wait_ep_pair(pb):
        pltpu.make_async_copy(ep_hbm.at[wid, 0], eb[pb], esem[pb]).wait()

    def issue_scatter(b, q):
        pltpu.async_copy(msg[b], aggr_sp.at[didx[q]], ssem[b], add=True)

    def wait_scatter(b):
        pltpu.make_async_copy(msg[b], aggr_sp.at[didx[0]], ssem[b]).wait()

    def compute(b, pb, half):
        xrb, ebb, msgb = xr[b], eb[pb], msg[b]
        base = half * (D // 2)

        def rows(r4, _):
            for u in range(4):
                r = r4 * 4 + u
                for g in range(4):
                    packed = ebb[r, pl.ds(base + g * 16, 16)]
                    lo, hi = plsc.unpack(
                        plsc.bitcast(packed, jnp.bfloat16),
                        format=plsc.PackFormat.INTERLEAVED)
                    sa = pl.ds(g * 16, 16)
                    sb = pl.ds(64 + g * 16, 16)
                    msgb[r, sa] = jnp.maximum(xrb[r, sa] + lo, 0.0)
                    msgb[r, sb] = jnp.maximum(xrb[r, sb] + hi, 0.0)
            return 0

        lax.fori_loop(0, CHUNK // 4, rows, 0)

    for t in range(4):
        unpack_idx(t, t)
    issue_gather(0, 0, 0)
    issue_ep_pair(0, 0)
    issue_gather(1, 1, 1)
    issue_ep_pair(1, 1)

    zeros16 = jnp.zeros((16,), jnp.float32)

    def zrow(r, _):
        for c in range(D // 16):
            msg0[r, pl.ds(c * 16, 16)] = zeros16
        return 0

    lax.fori_loop(0, CHUNK, zrow, 0)
    zbase = sid * ROWS_LO

    def zcopy(k, _):
        pltpu.sync_copy(msg0, aggr_sp.at[pl.ds(zbase + CHUNK * k, CHUNK)])
        return 0

    lax.fori_loop(0, 15, zcopy, 0)

    @pl.when(sid < NS - 1)
    def _():
        pltpu.sync_copy(msg0.at[pl.ds(0, 24)],
                        aggr_sp.at[pl.ds(zbase + 600, 24)])

    @pl.when(sid == NS - 1)
    def _():
        pltpu.sync_copy(msg0, aggr_sp.at[pl.ds(9960, CHUNK)])

    plsc.subcore_barrier()

    def step(j, b, q, pb, half, do_wait_scatter, do_issue, ep_wait, ep_issue):
        wait_gather(b)
        if ep_wait:
            wait_ep_pair(pb)
        if do_wait_scatter:
            wait_scatter(b)
        compute(b, pb, half)
        issue_scatter(b, q)
        if do_issue:
            q2 = (q + 2) % 4
            unpack_idx(j + 2, q2)
            issue_gather(j + 2, b, q2)
        if ep_issue:
            issue_ep_pair(j // 2 + 1, 1 - pb)

    wait_gather(0)
    wait_ep_pair(0)
    compute(0, 0, 0)
    issue_scatter(0, 0)
    issue_gather(2, 0, 2)

    wait_gather(1)
    compute(1, 0, 1)
    issue_scatter(1, 1)
    issue_gather(3, 1, 3)

    step(2, 0, 2, 1, 0, True, True, True, True)
    step(3, 1, 3, 1, 1, True, True, False, False)

    def quad(i, _):
        j = 4 * i
        step(j + 0, 0, 0, 0, 0, True, True, True, True)
        step(j + 1, 1, 1, 0, 1, True, True, False, False)
        step(j + 2, 0, 2, 1, 0, True, True, True, True)
        step(j + 3, 1, 3, 1, 1, True, True, False, False)
        return 0

    lax.fori_loop(1, 61, quad, 0)

    step(244, 0, 0, 0, 0, True, True, True, True)
    step(245, 1, 1, 0, 1, True, True, False, False)
    step(246, 0, 2, 1, 0, True, True, True, True)
    step(247, 1, 3, 1, 1, True, True, False, False)
    step(248, 0, 0, 0, 0, True, False, True, False)
    step(249, 1, 1, 0, 1, True, False, False, False)
    wait_scatter(0)
    wait_scatter(1)

    plsc.subcore_barrier()

    @pl.when(sid < NS - 1)
    def _():
        pltpu.sync_copy(aggr_sp.at[pl.ds(zbase, ROWS_LO)],
                        out_hbm.at[cid, pl.ds(zbase, ROWS_LO)])

    @pl.when(sid == NS - 1)
    def _():
        pltpu.sync_copy(aggr_sp.at[pl.ds((NS - 1) * ROWS_LO, ROWS_HI)],
                        out_hbm.at[cid, pl.ds((NS - 1) * ROWS_LO, ROWS_HI)])


def _aggregate_sc(x, e_proj, pidx):
    mesh = plsc.VectorSubcoreMesh(core_axis_name="c", subcore_axis_name="s")
    f = pl.kernel(
        _sc_body,
        out_type=jax.ShapeDtypeStruct((NC, N_NODES, D), jnp.float32),
        mesh=mesh,
        compiler_params=pltpu.CompilerParams(needs_layout_passes=False),
        scratch_types=[
            pltpu.VMEM((E_PER_W,), jnp.int32),
            pltpu.VMEM((CHUNK,), jnp.int32),
            pltpu.VMEM((CHUNK,), jnp.int32),
            pltpu.VMEM((CHUNK,), jnp.int32),
            pltpu.VMEM((CHUNK,), jnp.int32),
            pltpu.VMEM((CHUNK,), jnp.int32),
            pltpu.VMEM((CHUNK,), jnp.int32),
            pltpu.VMEM((CHUNK,), jnp.int32),
            pltpu.VMEM((CHUNK,), jnp.int32),
            pltpu.VMEM((CHUNK, D), jnp.float32),
            pltpu.VMEM((CHUNK, D), jnp.float32),
            pltpu.VMEM((CHUNK, D), jnp.int32),
            pltpu.VMEM((CHUNK, D), jnp.int32),
            pltpu.VMEM((CHUNK, D), jnp.float32),
            pltpu.VMEM((CHUNK, D), jnp.float32),
            pltpu.VMEM_SHARED((N_NODES, D), jnp.float32),
            pltpu.SemaphoreType.DMA,
            pltpu.SemaphoreType.DMA,
            pltpu.SemaphoreType.DMA,
            pltpu.SemaphoreType.DMA,
            pltpu.SemaphoreType.DMA,
            pltpu.SemaphoreType.DMA,
        ],
    )
    pidx2 = pidx.reshape(NW, E_PER_W)
    return f(x, e_proj, pidx2)


def _mlp_body(x_ref, a0_ref, a1_ref, w1_ref, b1_ref, w2_ref, b2_ref, out_ref):
    h = x_ref[...] + a0_ref[...] + a1_ref[...]
    h = jnp.dot(h, w1_ref[...], preferred_element_type=jnp.float32) + b1_ref[...]
    h = jnp.maximum(h, 0.0)
    out_ref[...] = (
        jnp.dot(h, w2_ref[...], preferred_element_type=jnp.float32) + b2_ref[...]
    )


def _mlp(x, a0, a1, W1, b1, W2, b2):
    BN = 2000
    full = lambda i: (0, 0)
    return pl.pallas_call(
        _mlp_body,
        grid=(N_NODES // BN,),
        in_specs=[
            pl.BlockSpec((BN, D), lambda i: (i, 0)),
            pl.BlockSpec((BN, D), lambda i: (i, 0)),
            pl.BlockSpec((BN, D), lambda i: (i, 0)),
            pl.BlockSpec((D, D), full),
            pl.BlockSpec((1, D), full),
            pl.BlockSpec((D, D), full),
            pl.BlockSpec((1, D), full),
        ],
        out_specs=pl.BlockSpec((BN, D), lambda i: (i, 0)),
        out_shape=jax.ShapeDtypeStruct((N_NODES, D), jnp.float32),
    )(x, a0, a1, W1, b1.reshape(1, D), W2, b2.reshape(1, D))


def kernel(x, edge_index, edge_attr, W_edge, b_edge, W1, b1, W2, b2):
    x = x.astype(jnp.float32)
    edge_attr = edge_attr.astype(jnp.float32)
    ei = edge_index.astype(jnp.int32)
    src, dst = ei[0], ei[1]
    pidx = src | (dst << 14)

    e_proj = _eproj(edge_attr, W_edge, b_edge)
    aggr = _aggregate_sc(x, e_proj, pidx)
    return _mlp(x, aggr[0], aggr[1], W1, b1, W2, b2)
